# baseline (speedup 1.0000x reference)
import functools
import math

import jax
import jax.numpy as jnp
from jax import lax
from jax.experimental import pallas as pl
from jax.experimental.pallas import tpu as pltpu
from jax.experimental.pallas import tpu_sc as plsc

D = 64
SCALE = math.sqrt(D)
L = 16
V = 1000000
NBLK = V // 128 + 1


def _mesh():
    return plsc.VectorSubcoreMesh(core_axis_name="c", subcore_axis_name="s")


@functools.partial(jax.jit, static_argnames=())
def _table_rows_sc(table_t):
    info = plsc.get_sparse_core_info()
    nc = info.num_cores
    nw = nc * info.num_subcores

    @functools.partial(
        pl.kernel,
        mesh=_mesh(),
        out_type=jax.ShapeDtypeStruct((V // 2, 128), jnp.float32),
        scratch_types=[
            pltpu.VMEM((64, 128), jnp.float32),
            pltpu.VMEM((64, 128), jnp.float32),
            pltpu.VMEM((64, 128), jnp.float32),
            pltpu.SemaphoreType.DMA,
            pltpu.SemaphoreType.DMA,
        ],
        compiler_params=pltpu.CompilerParams(needs_layout_passes=False),
    )
    def k(tab, out, tile0, tile1, tr_v, sem0, sem1):
        wid = lax.axis_index("s") * nc + lax.axis_index("c")
        per = NBLK // nw
        ext = NBLK % nw
        n = jnp.where(wid < ext, per + 1, per)
        base = wid * per + jnp.minimum(wid, ext)

        lane = lax.broadcasted_iota(jnp.int32, (L,), 0)

        def start_read(c, buf, sem):
            @pl.when(c < base + n)
            def _():
                pltpu.async_copy(tab.at[:, pl.ds(c * 128, 128)], buf, sem)

        def wait_read(c, buf, sem):
            pltpu.make_async_copy(tab.at[:, pl.ds(c * 128, 128)], buf, sem).wait()

        start_read(base, tile0, sem0)

        def body(i, carry):
            c = base + i
            cur = i % 2

            def do(cur_buf, cur_sem, nxt_buf, nxt_sem):
                wait_read(c, cur_buf, cur_sem)
                start_read(c + 1, nxt_buf, nxt_sem)
                def row_body(r, carry2):
                    for j in range(8):
                        d16 = (j % 4) * 16 + lane
                        lsp = lane * 0 + 2 * r + (1 if j >= 4 else 0)
                        vals = plsc.load_gather(cur_buf, [d16, lsp])
                        tr_v[r, pl.ds(j * 16, L)] = vals
                    return carry2

                lax.fori_loop(0, 64, row_body, 0, unroll=2)

                @pl.when(c < NBLK - 1)
                def _():
                    pltpu.sync_copy(tr_v, out.at[pl.ds(c * 64, 64)])

                @pl.when(c == NBLK - 1)
                def _():
                    pltpu.sync_copy(
                        tr_v.at[pl.ds(0, 32)], out.at[pl.ds(c * 64, 32)]
                    )

            lax.cond(
                cur == 0,
                lambda: do(tile0, sem0, tile1, sem1),
                lambda: do(tile1, sem1, tile0, sem0),
            )
            return carry

        lax.fori_loop(0, n, body, 0)

    return k(table_t)


@functools.partial(jax.jit, static_argnames=())
def _gather_sc(x_t, rows):
    info = plsc.get_sparse_core_info()
    nc = info.num_cores
    nw = nc * info.num_subcores
    n_groups = 200 * 32
    per_w = n_groups // nw
    B = 4
    n_batches = per_w // B

    @functools.partial(
        pl.kernel,
        mesh=_mesh(),
        out_type=jax.ShapeDtypeStruct((200, 8, 32, 8, 128), jnp.float32),
        scratch_types=[
            pltpu.VMEM((B * 128,), jnp.int32),
            pltpu.VMEM((B * 128,), jnp.int32),
            pltpu.VMEM((B * 128, D), jnp.float32),
            pltpu.VMEM((B * 128, D), jnp.float32),
            pltpu.VMEM((64, 128), jnp.float32),
            pltpu.SemaphoreType.DMA,
            pltpu.SemaphoreType.DMA,
        ],
        compiler_params=pltpu.CompilerParams(
            needs_layout_passes=False, use_tc_tiling_on_sc=False
        ),
    )
    def k(xt, tab, out, idx0, idx1, rows0, rows1, tr_v, sem0, sem1):
        wid = lax.axis_index("s") * nc + lax.axis_index("c")
        g0 = wid * per_w

        lane = lax.broadcasted_iota(jnp.int32, (L,), 0)

        def stage(bi, idx_buf, sem, rows_buf):
            @pl.when(bi < n_batches)
            def _():
                for q in range(B):
                    g = g0 + bi * B + q
                    t = g // 32
                    bt = g % 32
                    pltpu.sync_copy(
                        xt.at[t, pl.ds(bt * 128, 128)],
                        idx_buf.at[pl.ds(q * 128, 128)],
                    )
                pltpu.async_copy(tab.at[idx_buf], rows_buf, sem)

        stage(0, idx0, sem0, rows0)

        def body(bi, carry):
            def do(idx_cur, sem_cur, rows_cur, idx_nxt, sem_nxt, rows_nxt):
                pltpu.make_async_copy(tab.at[idx_cur], rows_cur, sem_cur).wait()
                stage(bi + 1, idx_nxt, sem_nxt, rows_nxt)
                for q in range(B):
                    g = g0 + bi * B + q
                    t = g // 32
                    bt = g % 32

                    def d_body(d, carry2):
                        dsp = lane * 0 + d
                        for lg in range(8):
                            lvec = q * 128 + lg * 16 + lane
                            vals = plsc.load_gather(rows_cur, [lvec, dsp])
                            tr_v[d, pl.ds(lg * 16, L)] = vals * SCALE
                        return carry2

                    lax.fori_loop(0, 64, d_body, 0, unroll=2)
                    for dt in range(8):
                        pltpu.sync_copy(
                            tr_v.at[pl.ds(dt * 8, 8)], out.at[t, dt, bt]
                        )

            lax.cond(
                bi % 2 == 0,
                lambda: do(idx0, sem0, rows0, idx1, sem1, rows1),
                lambda: do(idx1, sem1, rows1, idx0, sem0, rows0),
            )
            return carry

        lax.fori_loop(0, n_batches, body, 0)

    return k(x_t, rows)


def kernel(x, table):
    s, t = x.shape
    table_t = table.T
    rows2 = _table_rows_sc(table_t)
    rows = rows2.reshape(V, D)
    x_t = x.T
    out5 = _gather_sc(x_t, rows)
    return out5.transpose((2, 4, 0, 1, 3)).reshape(s, t, D)


# speedup vs baseline: 1.5427x; 1.5427x over previous
import functools
import math

import jax
import jax.numpy as jnp
from jax import lax
from jax.experimental import pallas as pl
from jax.experimental.pallas import tpu as pltpu
from jax.experimental.pallas import tpu_sc as plsc

D = 64
SCALE = math.sqrt(D)
L = 16
V = 1000000
NBLK = V // 128 + 1
PITCH = 136


def _mesh():
    return plsc.VectorSubcoreMesh(core_axis_name="c", subcore_axis_name="s")


@jax.jit
def _table_rows_sc(table_t):
    info = plsc.get_sparse_core_info()
    nc = info.num_cores
    nw = nc * info.num_subcores

    @functools.partial(
        pl.kernel,
        mesh=_mesh(),
        out_type=jax.ShapeDtypeStruct((V // 2, 128), jnp.float32),
        scratch_types=[
            pltpu.VMEM((64, PITCH), jnp.float32),
            pltpu.VMEM((64, PITCH), jnp.float32),
            pltpu.VMEM((64, 128), jnp.float32),
            pltpu.VMEM((64, 128), jnp.float32),
            pltpu.SemaphoreType.DMA,
            pltpu.SemaphoreType.DMA,
            pltpu.SemaphoreType.DMA,
        ],
        compiler_params=pltpu.CompilerParams(needs_layout_passes=False),
    )
    def k(tab, out, tile0, tile1, tr0, tr1, sem0, sem1, semw):
        wid = lax.axis_index("s") * nc + lax.axis_index("c")
        per = NBLK // nw
        ext = NBLK % nw
        n = jnp.where(wid < ext, per + 1, per)
        base = wid * per + jnp.minimum(wid, ext)
        is_last = wid == nw - 1
        n_loop = jnp.where(is_last, n - 1, n)

        lane = lax.broadcasted_iota(jnp.int32, (L,), 0)
        d16 = [(j % 4) * 16 + lane for j in range(8)]

        def read(c, buf, sem):
            return pltpu.make_async_copy(
                tab.at[:, pl.ds(c * 128, 128)], buf.at[:, pl.ds(0, 128)], sem
            )

        def transpose(buf, tr):
            def row_body(r, carry2):
                s0 = lane * 0 + 2 * r
                for j in range(8):
                    lsp = s0 + (1 if j >= 4 else 0)
                    vals = plsc.load_gather(buf, [d16[j], lsp])
                    tr[r, pl.ds(j * 16, L)] = vals
                return carry2

            lax.fori_loop(0, 64, row_body, 0, unroll=2)

        @pl.when(is_last)
        def _():
            c = NBLK - 1 + wid * 0
            read(c, tile0, sem0).start()
            read(c, tile0, sem0).wait()
            transpose(tile0, tr0)
            pltpu.sync_copy(tr0.at[pl.ds(0, 32)], out.at[pl.ds(c * 64, 32)])

        read(base, tile0, sem0).start()

        @pl.when(n_loop > 1)
        def _():
            read(base + 1, tile1, sem1).start()

        def body(i, carry):
            c = base + i

            def do(cur_buf, cur_sem, tr):
                read(c, cur_buf, cur_sem).wait()

                @pl.when(i >= 2)
                def _():
                    pltpu.make_async_copy(
                        tr, out.at[pl.ds((c - 2) * 64, 64)], semw
                    ).wait()

                transpose(cur_buf, tr)

                @pl.when(i + 2 < n_loop)
                def _():
                    read(c + 2, cur_buf, cur_sem).start()

                pltpu.async_copy(tr, out.at[pl.ds(c * 64, 64)], semw)

            lax.cond(
                i % 2 == 0,
                lambda: do(tile0, sem0, tr0),
                lambda: do(tile1, sem1, tr1),
            )
            return carry

        lax.fori_loop(0, n_loop, body, 0)

        def drain(i, carry):
            c = base + n_loop - 2 + i

            @pl.when(c >= base)
            def _():
                pltpu.make_async_copy(
                    tr0, out.at[pl.ds(c * 64, 64)], semw
                ).wait()

            return carry

        lax.fori_loop(0, 2, drain, 0)

    return k(table_t)


@jax.jit
def _gather_sc(x_t, rows):
    info = plsc.get_sparse_core_info()
    nc = info.num_cores
    nw = nc * info.num_subcores
    n_groups = 200 * 32
    per_w = n_groups // nw
    B = 4
    n_batches = per_w // B

    @functools.partial(
        pl.kernel,
        mesh=_mesh(),
        out_type=jax.ShapeDtypeStruct((200, 8, 32, 8, 128), jnp.float32),
        scratch_types=[
            pltpu.VMEM((B * 128,), jnp.int32),
            pltpu.VMEM((B * 128,), jnp.int32),
            pltpu.VMEM((B * 128, D), jnp.float32),
            pltpu.VMEM((B * 128, D), jnp.float32),
            pltpu.VMEM((64, PITCH), jnp.float32),
            pltpu.VMEM((64, PITCH), jnp.float32),
            pltpu.SemaphoreType.DMA,
            pltpu.SemaphoreType.DMA,
            pltpu.SemaphoreType.DMA,
        ],
        compiler_params=pltpu.CompilerParams(
            needs_layout_passes=False, use_tc_tiling_on_sc=False
        ),
    )
    def k(xt, tab, out, idx0, idx1, rows0, rows1, tr0, tr1, sem0, sem1, semw):
        wid = lax.axis_index("s") * nc + lax.axis_index("c")
        g0 = wid * per_w

        lane = lax.broadcasted_iota(jnp.int32, (L,), 0)
        row16 = [j * 16 + lane for j in range(4)]

        def stage(bi, idx_buf, sem, rows_buf):
            @pl.when(bi < n_batches)
            def _():
                g = g0 + bi * B
                t = g // 32
                bt = g % 32
                pltpu.sync_copy(xt.at[t, pl.ds(bt * 128, B * 128)], idx_buf)
                pltpu.async_copy(tab.at[idx_buf], rows_buf, sem)

        stage(0, idx0, sem0, rows0)

        def body(bi, carry):
            def do(idx_cur, sem_cur, rows_cur, idx_nxt, sem_nxt, rows_nxt):
                pltpu.make_async_copy(tab.at[idx_cur], rows_cur, sem_cur).wait()
                stage(bi + 1, idx_nxt, sem_nxt, rows_nxt)
                for q in range(B):
                    g = g0 + bi * B + q
                    t = g // 32
                    bt = g % 32
                    gi = bi * B + q
                    tr = tr0 if q % 2 == 0 else tr1

                    @pl.when(gi >= 2)
                    def _():
                        for dt in range(8):
                            pltpu.make_async_copy(
                                tr.at[pl.ds(dt * 8, 8), pl.ds(0, 128)],
                                out.at[t, dt, bt],
                                semw,
                            ).wait()

                    def tok_body(tok, carry2):
                        csp = lane * 0 + tok
                        for j in range(4):
                            vals = rows_cur[q * 128 + tok, pl.ds(j * 16, L)]
                            plsc.store_scatter(tr, [row16[j], csp], vals * SCALE)
                        return carry2

                    lax.fori_loop(0, 128, tok_body, 0, unroll=2)
                    for dt in range(8):
                        pltpu.async_copy(
                            tr.at[pl.ds(dt * 8, 8), pl.ds(0, 128)],
                            out.at[t, dt, bt],
                            semw,
                        )

            lax.cond(
                bi % 2 == 0,
                lambda: do(idx0, sem0, rows0, idx1, sem1, rows1),
                lambda: do(idx1, sem1, rows1, idx0, sem0, rows0),
            )
            return carry

        lax.fori_loop(0, n_batches, body, 0)

        for _q in range(2):
            for dt in range(8):
                pltpu.make_async_copy(
                    tr0.at[pl.ds(dt * 8, 8), pl.ds(0, 128)],
                    out.at[199, dt, 31],
                    semw,
                ).wait()

    return k(x_t, rows)


def kernel(x, table):
    s, t = x.shape
    table_t = table.T
    rows2 = _table_rows_sc(table_t)
    rows = rows2.reshape(V, D)
    x_t = x.T
    out5 = _gather_sc(x_t, rows)
    return out5.transpose((2, 4, 0, 1, 3)).reshape(s, t, D)


# speedup vs baseline: 2.5819x; 1.6736x over previous
import functools
import math

import jax
import jax.numpy as jnp
from jax import lax
from jax.experimental import pallas as pl
from jax.experimental.pallas import tpu as pltpu
from jax.experimental.pallas import tpu_sc as plsc

D = 64
SCALE = math.sqrt(D)
L = 16
V = 1000000
NBLK = V // 128 + 1
PITCH = 137


def _mesh():
    return plsc.VectorSubcoreMesh(core_axis_name="c", subcore_axis_name="s")


@jax.jit
def _table_rows_sc(table_t):
    info = plsc.get_sparse_core_info()
    nc = info.num_cores
    nw = nc * info.num_subcores

    @functools.partial(
        pl.kernel,
        mesh=_mesh(),
        out_type=jax.ShapeDtypeStruct((V // 2, 128), jnp.float32),
        scratch_types=[
            pltpu.VMEM((64, 128), jnp.float32),
            pltpu.VMEM((64, 128), jnp.float32),
            pltpu.VMEM((64, 128), jnp.float32),
            pltpu.VMEM((64, 128), jnp.float32),
            pltpu.SemaphoreType.DMA,
            pltpu.SemaphoreType.DMA,
            pltpu.SemaphoreType.DMA,
        ],
        compiler_params=pltpu.CompilerParams(needs_layout_passes=False),
    )
    def k(tab, out, tile0, tile1, tr0, tr1, sem0, sem1, semw):
        wid = lax.axis_index("s") * nc + lax.axis_index("c")
        per = NBLK // nw
        ext = NBLK % nw
        n = jnp.where(wid < ext, per + 1, per)
        base = wid * per + jnp.minimum(wid, ext)
        is_last = wid == nw - 1
        n_loop = jnp.where(is_last, n - 1, n)

        lane = lax.broadcasted_iota(jnp.int32, (L,), 0)
        dvec = [a * 16 + lane for a in range(4)]

        def read_start(c, buf, sem):
            for rt in range(8):
                pltpu.make_async_copy(
                    tab.at[pl.ds(rt * 8, 8), pl.ds(c * 128, 128)],
                    buf.at[pl.ds(rt * 8, 8)],
                    sem,
                ).start()

        def read_wait(c, buf, sem):
            for rt in range(8):
                pltpu.make_async_copy(
                    tab.at[pl.ds(rt * 8, 8), pl.ds(c * 128, 128)],
                    buf.at[pl.ds(rt * 8, 8)],
                    sem,
                ).wait()

        def transpose(buf, tr):
            def blk_body(b, carry2):
                for a in range(4):
                    dv = dvec[a]
                    for m in range(16):
                        rot = (lane + m) & 15
                        lvec = b * 16 + rot
                        vals = plsc.load_gather(buf, [dv, lvec])
                        row = lvec >> 1
                        col = ((rot & 1) << 6) + dv
                        plsc.store_scatter(tr, [row, col], vals)
                return carry2

            lax.fori_loop(0, 8, blk_body, 0)

        @pl.when(is_last)
        def _():
            c = NBLK - 1 + wid * 0
            read_start(c, tile0, sem0)
            read_wait(c, tile0, sem0)
            transpose(tile0, tr0)
            pltpu.sync_copy(tr0.at[pl.ds(0, 32)], out.at[pl.ds(c * 64, 32)])

        read_start(base, tile0, sem0)

        @pl.when(n_loop > 1)
        def _():
            read_start(base + 1, tile1, sem1)

        def body(i, carry):
            c = base + i

            def do(cur_buf, cur_sem, tr):
                read_wait(c, cur_buf, cur_sem)

                @pl.when(i >= 2)
                def _():
                    pltpu.make_async_copy(
                        tr, out.at[pl.ds((c - 2) * 64, 64)], semw
                    ).wait()

                transpose(cur_buf, tr)

                @pl.when(i + 2 < n_loop)
                def _():
                    read_start(c + 2, cur_buf, cur_sem)

                pltpu.async_copy(tr, out.at[pl.ds(c * 64, 64)], semw)

            lax.cond(
                i % 2 == 0,
                lambda: do(tile0, sem0, tr0),
                lambda: do(tile1, sem1, tr1),
            )
            return carry

        lax.fori_loop(0, n_loop, body, 0)

        def drain(i, carry):
            c = base + n_loop - 2 + i

            @pl.when(c >= base)
            def _():
                pltpu.make_async_copy(
                    tr0, out.at[pl.ds(c * 64, 64)], semw
                ).wait()

            return carry

        lax.fori_loop(0, 2, drain, 0)

    return k(table_t)


@jax.jit
def _gather_sc(x_flat, rows):
    info = plsc.get_sparse_core_info()
    nc = info.num_cores
    nw = nc * info.num_subcores
    n_groups = 200 * 32
    per_w = n_groups // nw
    B = 4
    n_batches = per_w // B

    @functools.partial(
        pl.kernel,
        mesh=_mesh(),
        out_type=jax.ShapeDtypeStruct((200, 8, 32, 8, 128), jnp.float32),
        scratch_types=[
            pltpu.VMEM((per_w * 128,), jnp.int32),
            pltpu.VMEM((B * 128, D), jnp.float32),
            pltpu.VMEM((B * 128, D), jnp.float32),
            pltpu.VMEM((64, PITCH), jnp.float32),
            pltpu.VMEM((64, PITCH), jnp.float32),
            pltpu.SemaphoreType.DMA,
            pltpu.SemaphoreType.DMA,
            pltpu.SemaphoreType.DMA,
        ],
        compiler_params=pltpu.CompilerParams(
            needs_layout_passes=False, use_tc_tiling_on_sc=False
        ),
    )
    def k(xf, tab, out, idx_all, rows0, rows1, tr0, tr1, sem0, sem1, semw):
        wid = lax.axis_index("s") * nc + lax.axis_index("c")
        g0 = wid * per_w

        lane = lax.broadcasted_iota(jnp.int32, (L,), 0)
        row16 = [j * 16 + lane for j in range(4)]

        pltpu.sync_copy(xf.at[pl.ds(g0 * 128, per_w * 128)], idx_all)

        def stage(bi, sem, rows_buf):
            @pl.when(bi < n_batches)
            def _():
                pltpu.async_copy(
                    tab.at[idx_all.at[pl.ds(bi * (B * 128), B * 128)]],
                    rows_buf,
                    sem,
                )

        def wait_stage(bi, sem, rows_buf):
            pltpu.make_async_copy(
                tab.at[idx_all.at[pl.ds(bi * (B * 128), B * 128)]],
                rows_buf,
                sem,
            ).wait()

        stage(0, sem0, rows0)

        def body(bi, carry):
            def do(sem_cur, rows_cur, sem_nxt, rows_nxt):
                wait_stage(bi, sem_cur, rows_cur)
                stage(bi + 1, sem_nxt, rows_nxt)
                for q in range(B):
                    g = g0 + bi * B + q
                    t = g // 32
                    bt = g % 32
                    gi = bi * B + q
                    tr = tr0 if q % 2 == 0 else tr1

                    @pl.when(gi >= 2)
                    def _():
                        for dt in range(8):
                            pltpu.make_async_copy(
                                tr.at[pl.ds(dt * 8, 8), pl.ds(0, 128)],
                                out.at[t, dt, bt],
                                semw,
                            ).wait()

                    def tok_body(tok, carry2):
                        csp = lane * 0 + tok
                        for j in range(4):
                            vals = rows_cur[q * 128 + tok, pl.ds(j * 16, L)]
                            plsc.store_scatter(tr, [row16[j], csp], vals * SCALE)
                        return carry2

                    lax.fori_loop(0, 128, tok_body, 0, unroll=2)
                    for dt in range(8):
                        pltpu.async_copy(
                            tr.at[pl.ds(dt * 8, 8), pl.ds(0, 128)],
                            out.at[t, dt, bt],
                            semw,
                        )

            lax.cond(
                bi % 2 == 0,
                lambda: do(sem0, rows0, sem1, rows1),
                lambda: do(sem1, rows1, sem0, rows0),
            )
            return carry

        lax.fori_loop(0, n_batches, body, 0)

        for _q in range(2):
            for dt in range(8):
                pltpu.make_async_copy(
                    tr0.at[pl.ds(dt * 8, 8), pl.ds(0, 128)],
                    out.at[199, dt, 31],
                    semw,
                ).wait()

    return k(x_flat, rows)


def kernel(x, table):
    s, t = x.shape
    table_t = table.T
    rows2 = _table_rows_sc(table_t)
    rows = rows2.reshape(V, D)
    x_flat = x.T.reshape(s * t)
    out5 = _gather_sc(x_flat, rows)
    return out5.transpose((2, 4, 0, 1, 3)).reshape(s, t, D)
